# Initial kernel scaffold; baseline (speedup 1.0000x reference)
import jax
import jax.numpy as jnp
from jax.experimental import pallas as pl


def kernel(x):
    raise NotImplementedError("write your pallas kernel here")


# speedup vs baseline: 1.9487x; 1.9487x over previous
import functools

import jax
import jax.numpy as jnp
import numpy as np
from jax import lax
from jax.experimental import pallas as pl
from jax.experimental.pallas import tpu as pltpu
from jax.experimental.pallas import tpu_sc as plsc

_TOKENS = 512
_N, _L, _D = 128, 1025, 96
_T1 = _TOKENS + 1
_PAD = 520
_CHUNK = 128

_ROT_A = (13, 15, 26, 6)
_ROT_B = (17, 29, 16, 24)


def _rotl(x, d):
    return ((x << np.uint32(d)) | (x >> np.uint32(32 - d))).astype(np.uint32)


def _threefry2x32(k0, k1, x0, x1):
    ks0, ks1 = np.uint32(k0), np.uint32(k1)
    ks2 = np.uint32(ks0 ^ ks1 ^ np.uint32(0x1BD11BDA))
    x0 = (x0 + ks0).astype(np.uint32)
    x1 = (x1 + ks1).astype(np.uint32)
    keys = (ks0, ks1, ks2)
    for i, rset in enumerate((_ROT_A, _ROT_B, _ROT_A, _ROT_B, _ROT_A)):
        for r in rset:
            x0 = (x0 + x1).astype(np.uint32)
            x1 = _rotl(x1, r)
            x1 = (x1 ^ x0).astype(np.uint32)
        x0 = (x0 + keys[(i + 1) % 3]).astype(np.uint32)
        x1 = (x1 + keys[(i + 2) % 3] + np.uint32(i + 1)).astype(np.uint32)
    return x0, x1


def _uniform(seed: int, shape) -> np.ndarray:
    n = int(np.prod(shape))
    idx = np.arange(n, dtype=np.uint64)
    b1, b2 = _threefry2x32(
        np.uint32(seed >> 32), np.uint32(seed & 0xFFFFFFFF),
        (idx >> np.uint64(32)).astype(np.uint32),
        (idx & np.uint64(0xFFFFFFFF)).astype(np.uint32),
    )
    bits = (b1 ^ b2).astype(np.uint32)
    fl = ((bits >> np.uint32(9)) | np.uint32(0x3F800000)).view(np.float32)
    return np.maximum(np.float32(0.0), fl - np.float32(1.0)).reshape(shape)


@functools.lru_cache(maxsize=1)
def _flat_indices() -> np.ndarray:
    scores = _uniform(42, (_N, _L - 1))
    patch = np.argsort(scores, axis=1, kind="stable")[:, :_TOKENS] + 1
    patch = np.sort(patch, axis=1).astype(np.int32)
    mask = np.concatenate(
        [np.zeros((_N, 1), np.int32), patch], axis=1)
    base = (np.arange(_N, dtype=np.int32) * _L)[:, None]
    flat = mask + base
    pad = np.broadcast_to(base, (_N, _PAD - _T1))
    return np.ascontiguousarray(np.concatenate([flat, pad], axis=1))


@functools.lru_cache(maxsize=1)
def _sc_gather():
    info = plsc.get_sparse_core_info()
    nc, ns = info.num_cores, info.num_subcores
    nw = nc * ns
    per_w = _N // nw
    mesh = plsc.VectorSubcoreMesh(core_axis_name="c", subcore_axis_name="s")

    @functools.partial(
        pl.kernel,
        mesh=mesh,
        out_type=jax.ShapeDtypeStruct((_N, _T1, _D), jnp.float32),
        scratch_types=[
            pltpu.VMEM((_PAD,), jnp.int32),
            pltpu.VMEM((_CHUNK, _D), jnp.float32),
            pltpu.SemaphoreType.DMA,
        ],
        compiler_params=pltpu.CompilerParams(use_tc_tiling_on_sc=False),
    )
    def gather_kernel(xf, idxf, out, idx_v, buf, sem):
        wid = lax.axis_index("s") * nc + lax.axis_index("c")
        for b in range(per_w):
            n = wid * per_w + b
            pltpu.sync_copy(idxf.at[n], idx_v)
            for ci in range(4):
                t0 = ci * _CHUNK
                pltpu.async_copy(
                    xf.at[idx_v.at[pl.ds(t0, _CHUNK)]], buf, sem
                ).wait()
                pltpu.sync_copy(buf, out.at[n, pl.ds(t0, _CHUNK)])
            pltpu.async_copy(
                xf.at[idx_v.at[pl.ds(4 * _CHUNK, 8)]], buf.at[pl.ds(0, 8)], sem
            ).wait()
            pltpu.sync_copy(buf.at[pl.ds(0, 1)], out.at[n, pl.ds(4 * _CHUNK, 1)])

    return gather_kernel


def kernel(x):
    n, l, d = x.shape
    xf = x.reshape(n * l, d)
    idxf = jnp.asarray(_flat_indices())
    return _sc_gather()(xf, idxf)


# speedup vs baseline: 2.0768x; 1.0657x over previous
import functools

import jax
import jax.numpy as jnp
import numpy as np
from jax import lax
from jax.experimental import pallas as pl
from jax.experimental.pallas import tpu as pltpu
from jax.experimental.pallas import tpu_sc as plsc

_TOKENS = 512
_N, _L, _D = 128, 1025, 96
_T1 = _TOKENS + 1
_PAD = 520
_CHUNK = 128
_DP = 128
_SLOTS = 4

_ROT_A = (13, 15, 26, 6)
_ROT_B = (17, 29, 16, 24)


def _rotl(x, d):
    return ((x << np.uint32(d)) | (x >> np.uint32(32 - d))).astype(np.uint32)


def _threefry2x32(k0, k1, x0, x1):
    ks0, ks1 = np.uint32(k0), np.uint32(k1)
    ks2 = np.uint32(ks0 ^ ks1 ^ np.uint32(0x1BD11BDA))
    x0 = (x0 + ks0).astype(np.uint32)
    x1 = (x1 + ks1).astype(np.uint32)
    keys = (ks0, ks1, ks2)
    for i, rset in enumerate((_ROT_A, _ROT_B, _ROT_A, _ROT_B, _ROT_A)):
        for r in rset:
            x0 = (x0 + x1).astype(np.uint32)
            x1 = _rotl(x1, r)
            x1 = (x1 ^ x0).astype(np.uint32)
        x0 = (x0 + keys[(i + 1) % 3]).astype(np.uint32)
        x1 = (x1 + keys[(i + 2) % 3] + np.uint32(i + 1)).astype(np.uint32)
    return x0, x1


def _uniform(seed: int, shape) -> np.ndarray:
    n = int(np.prod(shape))
    idx = np.arange(n, dtype=np.uint64)
    b1, b2 = _threefry2x32(
        np.uint32(seed >> 32), np.uint32(seed & 0xFFFFFFFF),
        (idx >> np.uint64(32)).astype(np.uint32),
        (idx & np.uint64(0xFFFFFFFF)).astype(np.uint32),
    )
    bits = (b1 ^ b2).astype(np.uint32)
    fl = ((bits >> np.uint32(9)) | np.uint32(0x3F800000)).view(np.float32)
    return np.maximum(np.float32(0.0), fl - np.float32(1.0)).reshape(shape)


@functools.lru_cache(maxsize=1)
def _flat_indices() -> np.ndarray:
    scores = _uniform(42, (_N, _L - 1))
    patch = np.argsort(scores, axis=1, kind="stable")[:, :_TOKENS] + 1
    patch = np.sort(patch, axis=1).astype(np.int32)
    mask = np.concatenate(
        [np.zeros((_N, 1), np.int32), patch], axis=1)
    base = (np.arange(_N, dtype=np.int32) * _L)[:, None]
    flat = mask + base
    pad = np.broadcast_to(base, (_N, _PAD - _T1))
    return np.ascontiguousarray(np.concatenate([flat, pad], axis=1))


@functools.lru_cache(maxsize=1)
def _sc_gather():
    info = plsc.get_sparse_core_info()
    nc, ns = info.num_cores, info.num_subcores
    nw = nc * ns
    per_w = _N // nw
    mesh = plsc.VectorSubcoreMesh(core_axis_name="c", subcore_axis_name="s")

    sizes = [(_CHUNK, _CHUNK)] * 4 + [(8, 1)]

    @functools.partial(
        pl.kernel,
        mesh=mesh,
        out_type=jax.ShapeDtypeStruct((_N, _T1, _DP), jnp.float32),
        scratch_types=[
            pltpu.VMEM((2, _PAD), jnp.int32),
            pltpu.VMEM((_SLOTS, _CHUNK, _DP), jnp.float32),
            [pltpu.SemaphoreType.DMA] * _SLOTS,
            [pltpu.SemaphoreType.DMA] * _SLOTS,
        ],
    )
    def gather_kernel(xf, idxf, out, idx_v, gbuf, gsems, wsems):
        wid = lax.axis_index("s") * nc + lax.axis_index("c")
        steps = []
        for b in range(per_w):
            for ci in range(5):
                steps.append((b, ci))

        gather_pend = {}
        write_pend = {}

        def issue(k):
            b, ci = steps[k]
            slot = k % _SLOTS
            n = wid * per_w + b
            if ci == 0:
                pltpu.sync_copy(idxf.at[n], idx_v.at[b % 2])
            if k >= _SLOTS:
                write_pend.pop(k - _SLOTS).wait()
            gsz, _ = sizes[ci]
            gather_pend[k] = pltpu.async_copy(
                xf.at[idx_v.at[b % 2].at[pl.ds(ci * _CHUNK, gsz)]],
                gbuf.at[slot].at[pl.ds(0, gsz)], gsems[slot])

        def retire(k):
            b, ci = steps[k]
            slot = k % _SLOTS
            n = wid * per_w + b
            gather_pend.pop(k).wait()
            _, wsz = sizes[ci]
            write_pend[k] = pltpu.async_copy(
                gbuf.at[slot].at[pl.ds(0, wsz)],
                out.at[n, pl.ds(ci * _CHUNK, wsz)], wsems[slot])

        nsteps = len(steps)
        for k in range(min(_SLOTS - 1, nsteps)):
            issue(k)
        for k in range(nsteps):
            if k + _SLOTS - 1 < nsteps:
                issue(k + _SLOTS - 1)
            retire(k)
        for k in sorted(write_pend):
            write_pend.pop(k).wait()

    return gather_kernel


def kernel(x):
    n, l, d = x.shape
    xp = jnp.pad(x, ((0, 0), (0, 0), (0, _DP - d))).reshape(n * l, _DP)
    idxf = jnp.asarray(_flat_indices())
    outp = _sc_gather()(xp, idxf)
    return outp[:, :, :d]


# speedup vs baseline: 2.7961x; 1.3464x over previous
import functools

import jax
import jax.numpy as jnp
import numpy as np
from jax import lax
from jax.experimental import pallas as pl
from jax.experimental.pallas import tpu as pltpu
from jax.experimental.pallas import tpu_sc as plsc

_TOKENS = 512
_N, _L, _D = 128, 1025, 96
_T1 = _TOKENS + 1
_PAD = 520
_CHUNK = 128
_DP = 128
_SLOTS = 4

_ROT_A = (13, 15, 26, 6)
_ROT_B = (17, 29, 16, 24)


def _rotl(x, d):
    return ((x << np.uint32(d)) | (x >> np.uint32(32 - d))).astype(np.uint32)


def _threefry2x32(k0, k1, x0, x1):
    ks0, ks1 = np.uint32(k0), np.uint32(k1)
    ks2 = np.uint32(ks0 ^ ks1 ^ np.uint32(0x1BD11BDA))
    x0 = (x0 + ks0).astype(np.uint32)
    x1 = (x1 + ks1).astype(np.uint32)
    keys = (ks0, ks1, ks2)
    for i, rset in enumerate((_ROT_A, _ROT_B, _ROT_A, _ROT_B, _ROT_A)):
        for r in rset:
            x0 = (x0 + x1).astype(np.uint32)
            x1 = _rotl(x1, r)
            x1 = (x1 ^ x0).astype(np.uint32)
        x0 = (x0 + keys[(i + 1) % 3]).astype(np.uint32)
        x1 = (x1 + keys[(i + 2) % 3] + np.uint32(i + 1)).astype(np.uint32)
    return x0, x1


def _uniform(seed: int, shape) -> np.ndarray:
    n = int(np.prod(shape))
    idx = np.arange(n, dtype=np.uint64)
    b1, b2 = _threefry2x32(
        np.uint32(seed >> 32), np.uint32(seed & 0xFFFFFFFF),
        (idx >> np.uint64(32)).astype(np.uint32),
        (idx & np.uint64(0xFFFFFFFF)).astype(np.uint32),
    )
    bits = (b1 ^ b2).astype(np.uint32)
    fl = ((bits >> np.uint32(9)) | np.uint32(0x3F800000)).view(np.float32)
    return np.maximum(np.float32(0.0), fl - np.float32(1.0)).reshape(shape)


@functools.lru_cache(maxsize=1)
def _flat_indices() -> np.ndarray:
    scores = _uniform(42, (_N, _L - 1))
    patch = np.argsort(scores, axis=1, kind="stable")[:, :_TOKENS] + 1
    patch = np.sort(patch, axis=1).astype(np.int32)
    mask = np.concatenate(
        [np.zeros((_N, 1), np.int32), patch], axis=1)
    base = (np.arange(_N, dtype=np.int32) * _L)[:, None]
    flat = mask + base
    pad = np.broadcast_to(base, (_N, _PAD - _T1))
    return np.ascontiguousarray(np.concatenate([flat, pad], axis=1))


@functools.lru_cache(maxsize=1)
def _sc_gather():
    info = plsc.get_sparse_core_info()
    nc, ns = info.num_cores, info.num_subcores
    nw = nc * ns
    per_w = _N // nw
    mesh = plsc.VectorSubcoreMesh(core_axis_name="c", subcore_axis_name="s")

    sizes = [(_CHUNK, _CHUNK)] * 4 + [(8, 1)]

    @functools.partial(
        pl.kernel,
        mesh=mesh,
        out_type=jax.ShapeDtypeStruct((_N, _T1, _DP), jnp.float32),
        scratch_types=[
            pltpu.VMEM((2, _PAD), jnp.int32),
            pltpu.VMEM((_SLOTS, _CHUNK, _DP), jnp.float32),
            [pltpu.SemaphoreType.DMA] * _SLOTS,
            [pltpu.SemaphoreType.DMA] * _SLOTS,
        ],
    )
    def gather_kernel(xf, idxf, out, idx_v, gbuf, gsems, wsems):
        wid = lax.axis_index("s") * nc + lax.axis_index("c")
        steps = []
        for b in range(per_w):
            for ci in range(5):
                steps.append((b, ci))

        gather_pend = {}
        write_pend = {}

        def issue(k):
            b, ci = steps[k]
            slot = k % _SLOTS
            n = wid * per_w + b
            if ci == 0:
                pltpu.sync_copy(idxf.at[n], idx_v.at[b % 2])
            if k >= _SLOTS:
                write_pend.pop(k - _SLOTS).wait()
            gsz, _ = sizes[ci]
            gather_pend[k] = pltpu.async_copy(
                xf.at[idx_v.at[b % 2].at[pl.ds(ci * _CHUNK, gsz)]],
                gbuf.at[slot].at[pl.ds(0, gsz)], gsems[slot])

        def retire(k):
            b, ci = steps[k]
            slot = k % _SLOTS
            n = wid * per_w + b
            gather_pend.pop(k).wait()
            _, wsz = sizes[ci]
            write_pend[k] = pltpu.async_copy(
                gbuf.at[slot].at[pl.ds(0, wsz)],
                out.at[n, pl.ds(ci * _CHUNK, wsz)], wsems[slot])

        nsteps = len(steps)
        for k in range(min(_SLOTS - 1, nsteps)):
            issue(k)
        for k in range(nsteps):
            if k + _SLOTS - 1 < nsteps:
                issue(k + _SLOTS - 1)
            retire(k)
        for k in sorted(write_pend):
            write_pend.pop(k).wait()

    return gather_kernel


@functools.lru_cache(maxsize=1)
def _tc_pad():
    blk = 8

    @functools.partial(
        pl.pallas_call,
        grid=(_N // blk,),
        in_specs=[pl.BlockSpec((blk, _L, _D), lambda i: (i, 0, 0))],
        out_specs=pl.BlockSpec((blk * _L, _DP), lambda i: (i, 0)),
        out_shape=jax.ShapeDtypeStruct((_N * _L, _DP), jnp.float32),
    )
    def pad_kernel(x_ref, o_ref):
        for b in range(blk):
            o_ref[pl.ds(b * _L, _L), pl.ds(0, _D)] = x_ref[b]

    return pad_kernel


def kernel(x):
    n, l, d = x.shape
    xp = _tc_pad()(x)
    idxf = jnp.asarray(_flat_indices())
    outp = _sc_gather()(xp, idxf)
    return outp[:, :, :d]
